# baseline (speedup 1.0000x reference)
import jax
import jax.numpy as jnp
from jax.experimental import pallas as pl
from jax.experimental.pallas import tpu as pltpu


def _bn(h, g, b):
    mu = jnp.mean(h, axis=0)
    var = jnp.var(h, axis=0)
    return g * (h - mu) / jnp.sqrt(var + 1e-5) + b


def _seg_mean(vals, idx, num):
    s = jax.ops.segment_sum(vals, idx, num_segments=num)
    c = jax.ops.segment_sum(jnp.ones((vals.shape[0], 1), vals.dtype), idx, num_segments=num)
    return s / jnp.maximum(c, 1.0)


def _final_mlp_kernel(g_ref, w1_ref, b1_ref, w2_ref, b2_ref, out_ref):
    g = g_ref[...]
    h = jnp.maximum(jnp.dot(g, w1_ref[...], preferred_element_type=jnp.float32) + b1_ref[...], 0.0)
    out_ref[...] = jnp.dot(h, w2_ref[...], preferred_element_type=jnp.float32) + b2_ref[...]


def _final_mlp(g, w1, b1, w2, b2):
    B = g.shape[0]
    OUT = w2.shape[1]
    return pl.pallas_call(
        _final_mlp_kernel,
        out_shape=jax.ShapeDtypeStruct((B, OUT), jnp.float32),
    )(g, w1, b1.reshape(1, -1), w2, b2.reshape(1, -1))


def kernel(x, edge_attr, params, edge_index, batch, fragments, fragments_edge_index, fragments_batch):
    p = params
    n = x.shape[0]
    f = fragments.shape[0]
    B = 256
    L = 4
    xa = x @ p['W_atom'] + p['b_atom']
    xf = p['frag_emb'][fragments]
    arow = fragments_edge_index[0]
    fcol = fragments_edge_index[1]
    xa = xa + _seg_mean(xf[fcol], arow, n)
    src = edge_index[0]
    dst = edge_index[1]
    for i in range(L):
        e = edge_attr @ p['W_bond'][i] + p['b_bond'][i]
        m = jax.nn.relu(xa[src] + e)
        agg = jax.ops.segment_sum(m, dst, num_segments=n)
        h = (1.0 + p['eps_a'][i]) * xa + agg
        h = h @ p['a_W1'][i] + p['a_b1'][i]
        h = jax.nn.relu(_bn(h, p['a_g1'][i], p['a_be1'][i]))
        h = h @ p['a_W2'][i] + p['a_b2'][i]
        xa = jax.nn.relu(_bn(h, p['a_gbn'][i], p['a_bbn'][i]))
        msg = jax.nn.relu(xa @ p['a2f_W'][i] + p['a2f_b'][i])
        xf = xf + _seg_mean(msg[arow], fcol, f)
        hf = (1.0 + p['eps_f'][i]) * xf
        hf = hf @ p['f_W1'][i] + p['f_b1'][i]
        hf = jax.nn.relu(_bn(hf, p['f_g1'][i], p['f_be1'][i]))
        hf = hf @ p['f_W2'][i] + p['f_b2'][i]
        xf = jax.nn.relu(_bn(hf, p['f_gbn'][i], p['f_bbn'][i]))
        msg2 = jax.nn.relu(xf @ p['f2a_W'][i] + p['f2a_b'][i])
        xa = xa + _seg_mean(msg2[fcol], arow, n)
    xf_o = xf
    for j in range(2):
        xf_o = jax.nn.relu(xf_o @ p['fo_W'][j] + p['fo_b'][j])
    xa_o = xa
    for j in range(2):
        xa_o = jax.nn.relu(xa_o @ p['ao_W'][j] + p['ao_b'][j])
    gf = _seg_mean(xf_o, fragments_batch, B)
    ga = _seg_mean(xa_o, batch, B)
    g = ga + gf
    return _final_mlp(g, p['o_W1'], p['o_b1'], p['o_W2'], p['o_b2'])


# speedup vs baseline: 2.2724x; 2.2724x over previous
import functools

import jax
import jax.numpy as jnp
from jax import lax
from jax.experimental import pallas as pl
from jax.experimental.pallas import tpu as pltpu
from jax.experimental.pallas import tpu_sc as plsc

NC = 2
NS = 16
CHUNK = 128


def _sc_edge_kernel(xa_hbm, e_hbm, src_hbm, dst_hbm, out_hbm,
                    srcbuf, dstbuf, xbuf, ebuf, zbuf, acc):
    n_pad = acc.shape[0]
    n_edges = e_hbm.shape[0]
    total_chunks = n_edges // CHUNK
    nworkers = NC * NS
    c = lax.axis_index("c")
    s = lax.axis_index("s")
    wid = s * NC + c

    def _zrow(i, _):
        for j in range(8):
            zbuf[i, pl.ds(j * 16, 16)] = jnp.zeros((16,), jnp.float32)
        return 0
    lax.fori_loop(0, CHUNK, _zrow, 0)
    rows_per_tile = n_pad // NS
    base = s * rows_per_tile
    nfull = rows_per_tile // CHUNK
    for k in range(nfull):
        pltpu.sync_copy(zbuf, acc.at[pl.ds(base + k * CHUNK, CHUNK)])
    rem = rows_per_tile - nfull * CHUNK
    if rem:
        pltpu.sync_copy(zbuf.at[pl.ds(0, rem)],
                        acc.at[pl.ds(base + nfull * CHUNK, rem)])
    plsc.subcore_barrier()

    n_my = (total_chunks - wid + nworkers - 1) // nworkers

    def _chunk(i, _):
        off = (wid + i * nworkers) * CHUNK
        pltpu.sync_copy(src_hbm.at[pl.ds(off, CHUNK)], srcbuf)
        pltpu.sync_copy(dst_hbm.at[pl.ds(off, CHUNK)], dstbuf)
        pltpu.sync_copy(xa_hbm.at[srcbuf], xbuf)
        pltpu.sync_copy(e_hbm.at[pl.ds(off, CHUNK)], ebuf)

        def _row(r, _):
            for j in range(8):
                sl = pl.ds(j * 16, 16)
                xbuf[r, sl] = jnp.maximum(xbuf[r, sl] + ebuf[r, sl], 0.0)
            return 0
        lax.fori_loop(0, CHUNK, _row, 0)
        pltpu.sync_copy(xbuf, acc.at[dstbuf], add=True)
        return 0

    lax.fori_loop(0, n_my, _chunk, 0)
    plsc.subcore_barrier()
    pltpu.sync_copy(acc.at[pl.ds(base, rows_per_tile)],
                    out_hbm.at[c, pl.ds(base, rows_per_tile)])


def _round_up(v, m):
    return (v + m - 1) // m * m


def _sc_edge_agg(xa, e, src, dst):
    n = xa.shape[0]
    h = xa.shape[1]
    n_pad = _round_up(n, NS * 8)
    mesh = plsc.VectorSubcoreMesh(core_axis_name="c", subcore_axis_name="s",
                                  num_cores=NC, num_subcores=NS)
    fn = pl.kernel(
        _sc_edge_kernel,
        out_type=jax.ShapeDtypeStruct((NC, n_pad, h), jnp.float32),
        mesh=mesh,
        scratch_types=[
            pltpu.VMEM((CHUNK,), jnp.int32),
            pltpu.VMEM((CHUNK,), jnp.int32),
            pltpu.VMEM((CHUNK, h), jnp.float32),
            pltpu.VMEM((CHUNK, h), jnp.float32),
            pltpu.VMEM((CHUNK, h), jnp.float32),
            pltpu.VMEM_SHARED((n_pad, h), jnp.float32),
        ],
    )
    return fn(xa, e, src, dst)[:, :n]


def _bn(h, g, b):
    mu = jnp.mean(h, axis=0)
    var = jnp.var(h, axis=0)
    return g * (h - mu) / jnp.sqrt(var + 1e-5) + b


def _seg_mean(vals, idx, num):
    s = jax.ops.segment_sum(vals, idx, num_segments=num)
    c = jax.ops.segment_sum(jnp.ones((vals.shape[0], 1), vals.dtype), idx, num_segments=num)
    return s / jnp.maximum(c, 1.0)


def _final_mlp_kernel(g_ref, w1_ref, b1_ref, w2_ref, b2_ref, out_ref):
    g = g_ref[...]
    h = jnp.maximum(jnp.dot(g, w1_ref[...], preferred_element_type=jnp.float32) + b1_ref[...], 0.0)
    out_ref[...] = jnp.dot(h, w2_ref[...], preferred_element_type=jnp.float32) + b2_ref[...]


def _final_mlp(g, w1, b1, w2, b2):
    B = g.shape[0]
    OUT = w2.shape[1]
    return pl.pallas_call(
        _final_mlp_kernel,
        out_shape=jax.ShapeDtypeStruct((B, OUT), jnp.float32),
    )(g, w1, b1.reshape(1, -1), w2, b2.reshape(1, -1))


def kernel(x, edge_attr, params, edge_index, batch, fragments, fragments_edge_index, fragments_batch):
    p = params
    n = x.shape[0]
    f = fragments.shape[0]
    B = 256
    L = 4
    xa = x @ p['W_atom'] + p['b_atom']
    xf = p['frag_emb'][fragments]
    arow = fragments_edge_index[0]
    fcol = fragments_edge_index[1]
    xa = xa + _seg_mean(xf[fcol], arow, n)
    src = edge_index[0]
    dst = edge_index[1]
    for i in range(L):
        e = edge_attr @ p['W_bond'][i] + p['b_bond'][i]
        parts = _sc_edge_agg(xa, e, src, dst)
        agg = parts[0] + parts[1]
        h = (1.0 + p['eps_a'][i]) * xa + agg
        h = h @ p['a_W1'][i] + p['a_b1'][i]
        h = jax.nn.relu(_bn(h, p['a_g1'][i], p['a_be1'][i]))
        h = h @ p['a_W2'][i] + p['a_b2'][i]
        xa = jax.nn.relu(_bn(h, p['a_gbn'][i], p['a_bbn'][i]))
        msg = jax.nn.relu(xa @ p['a2f_W'][i] + p['a2f_b'][i])
        xf = xf + _seg_mean(msg[arow], fcol, f)
        hf = (1.0 + p['eps_f'][i]) * xf
        hf = hf @ p['f_W1'][i] + p['f_b1'][i]
        hf = jax.nn.relu(_bn(hf, p['f_g1'][i], p['f_be1'][i]))
        hf = hf @ p['f_W2'][i] + p['f_b2'][i]
        xf = jax.nn.relu(_bn(hf, p['f_gbn'][i], p['f_bbn'][i]))
        msg2 = jax.nn.relu(xf @ p['f2a_W'][i] + p['f2a_b'][i])
        xa = xa + _seg_mean(msg2[fcol], arow, n)
    xf_o = xf
    for j in range(2):
        xf_o = jax.nn.relu(xf_o @ p['fo_W'][j] + p['fo_b'][j])
    xa_o = xa
    for j in range(2):
        xa_o = jax.nn.relu(xa_o @ p['ao_W'][j] + p['ao_b'][j])
    gf = _seg_mean(xf_o, fragments_batch, B)
    ga = _seg_mean(xa_o, batch, B)
    g = ga + gf
    return _final_mlp(g, p['o_W1'], p['o_b1'], p['o_W2'], p['o_b2'])


# speedup vs baseline: 2.9721x; 1.3079x over previous
import functools

import jax
import jax.numpy as jnp
from jax import lax
from jax.experimental import pallas as pl
from jax.experimental.pallas import tpu as pltpu
from jax.experimental.pallas import tpu_sc as plsc

NC = 2
NS = 16
CHUNK = 128


def _sc_edge_kernel(xa_hbm, e_hbm, src_hbm, dst_hbm, out_hbm,
                    srcbuf, dstbuf, xbuf, ebuf, zbuf, acc):
    n_pad = acc.shape[0]
    n_edges = e_hbm.shape[0]
    total_chunks = n_edges // CHUNK
    nworkers = NC * NS
    c = lax.axis_index("c")
    s = lax.axis_index("s")
    wid = s * NC + c

    def _zrow(i, _):
        for j in range(8):
            zbuf[i, pl.ds(j * 16, 16)] = jnp.zeros((16,), jnp.float32)
        return 0
    lax.fori_loop(0, CHUNK, _zrow, 0)
    rows_per_tile = n_pad // NS
    base = s * rows_per_tile
    nfull = rows_per_tile // CHUNK
    for k in range(nfull):
        pltpu.sync_copy(zbuf, acc.at[pl.ds(base + k * CHUNK, CHUNK)])
    rem = rows_per_tile - nfull * CHUNK
    if rem:
        pltpu.sync_copy(zbuf.at[pl.ds(0, rem)],
                        acc.at[pl.ds(base + nfull * CHUNK, rem)])
    plsc.subcore_barrier()

    n_my = (total_chunks - wid + nworkers - 1) // nworkers

    def _chunk(i, _):
        off = (wid + i * nworkers) * CHUNK
        pltpu.sync_copy(src_hbm.at[pl.ds(off, CHUNK)], srcbuf)
        pltpu.sync_copy(dst_hbm.at[pl.ds(off, CHUNK)], dstbuf)
        pltpu.sync_copy(xa_hbm.at[srcbuf], xbuf)
        pltpu.sync_copy(e_hbm.at[pl.ds(off, CHUNK)], ebuf)

        def _row(r, _):
            for j in range(8):
                sl = pl.ds(j * 16, 16)
                xbuf[r, sl] = jnp.maximum(xbuf[r, sl] + ebuf[r, sl], 0.0)
            return 0
        lax.fori_loop(0, CHUNK, _row, 0)
        pltpu.sync_copy(xbuf, acc.at[dstbuf], add=True)
        return 0

    lax.fori_loop(0, n_my, _chunk, 0)
    plsc.subcore_barrier()
    pltpu.sync_copy(acc.at[pl.ds(base, rows_per_tile)],
                    out_hbm.at[c, pl.ds(base, rows_per_tile)])


def _round_up(v, m):
    return (v + m - 1) // m * m


def _sc_gs_kernel(vals_hbm, gidx_hbm, sidx_hbm, out_hbm,
                  gbuf, sbuf, xbuf, zbuf, acc):
    n_pad = acc.shape[0]
    k_pad = gidx_hbm.shape[0]
    total_chunks = k_pad // CHUNK
    nworkers = NC * NS
    c = lax.axis_index("c")
    s = lax.axis_index("s")
    wid = s * NC + c

    def _zrow(i, _):
        for j in range(8):
            zbuf[i, pl.ds(j * 16, 16)] = jnp.zeros((16,), jnp.float32)
        return 0
    lax.fori_loop(0, CHUNK, _zrow, 0)
    rows_per_tile = n_pad // NS
    base = s * rows_per_tile
    nfull = rows_per_tile // CHUNK
    for k in range(nfull):
        pltpu.sync_copy(zbuf, acc.at[pl.ds(base + k * CHUNK, CHUNK)])
    rem = rows_per_tile - nfull * CHUNK
    if rem:
        pltpu.sync_copy(zbuf.at[pl.ds(0, rem)],
                        acc.at[pl.ds(base + nfull * CHUNK, rem)])
    plsc.subcore_barrier()

    n_my = (total_chunks - wid + nworkers - 1) // nworkers

    def _chunk(i, _):
        off = (wid + i * nworkers) * CHUNK
        pltpu.sync_copy(gidx_hbm.at[pl.ds(off, CHUNK)], gbuf)
        pltpu.sync_copy(sidx_hbm.at[pl.ds(off, CHUNK)], sbuf)
        pltpu.sync_copy(vals_hbm.at[gbuf], xbuf)
        pltpu.sync_copy(xbuf, acc.at[sbuf], add=True)
        return 0

    lax.fori_loop(0, n_my, _chunk, 0)
    plsc.subcore_barrier()
    pltpu.sync_copy(acc.at[pl.ds(base, rows_per_tile)],
                    out_hbm.at[c, pl.ds(base, rows_per_tile)])


def _sc_gather_scatter(vals, gidx, sidx, r):
    h = vals.shape[1]
    k = gidx.shape[0]
    k_pad = _round_up(k, CHUNK)
    n_pad = _round_up(r + 1, NS * 8)
    if k_pad != k:
        gidx = jnp.concatenate([gidx, jnp.zeros((k_pad - k,), jnp.int32)])
        sidx = jnp.concatenate([sidx, jnp.full((k_pad - k,), r, jnp.int32)])
    mesh = plsc.VectorSubcoreMesh(core_axis_name="c", subcore_axis_name="s",
                                  num_cores=NC, num_subcores=NS)
    fn = pl.kernel(
        _sc_gs_kernel,
        out_type=jax.ShapeDtypeStruct((NC, n_pad, h), jnp.float32),
        mesh=mesh,
        scratch_types=[
            pltpu.VMEM((CHUNK,), jnp.int32),
            pltpu.VMEM((CHUNK,), jnp.int32),
            pltpu.VMEM((CHUNK, h), jnp.float32),
            pltpu.VMEM((CHUNK, h), jnp.float32),
            pltpu.VMEM_SHARED((n_pad, h), jnp.float32),
        ],
    )
    parts = fn(vals, gidx, sidx)
    return (parts[0] + parts[1])[:r]


def _sc_edge_agg(xa, e, src, dst):
    n = xa.shape[0]
    h = xa.shape[1]
    n_pad = _round_up(n, NS * 8)
    mesh = plsc.VectorSubcoreMesh(core_axis_name="c", subcore_axis_name="s",
                                  num_cores=NC, num_subcores=NS)
    fn = pl.kernel(
        _sc_edge_kernel,
        out_type=jax.ShapeDtypeStruct((NC, n_pad, h), jnp.float32),
        mesh=mesh,
        scratch_types=[
            pltpu.VMEM((CHUNK,), jnp.int32),
            pltpu.VMEM((CHUNK,), jnp.int32),
            pltpu.VMEM((CHUNK, h), jnp.float32),
            pltpu.VMEM((CHUNK, h), jnp.float32),
            pltpu.VMEM((CHUNK, h), jnp.float32),
            pltpu.VMEM_SHARED((n_pad, h), jnp.float32),
        ],
    )
    return fn(xa, e, src, dst)[:, :n]


def _bn(h, g, b):
    mu = jnp.mean(h, axis=0)
    var = jnp.var(h, axis=0)
    return g * (h - mu) / jnp.sqrt(var + 1e-5) + b


def _seg_mean(vals, idx, num):
    s = jax.ops.segment_sum(vals, idx, num_segments=num)
    c = jax.ops.segment_sum(jnp.ones((vals.shape[0], 1), vals.dtype), idx, num_segments=num)
    return s / jnp.maximum(c, 1.0)


def _final_mlp_kernel(g_ref, w1_ref, b1_ref, w2_ref, b2_ref, out_ref):
    g = g_ref[...]
    h = jnp.maximum(jnp.dot(g, w1_ref[...], preferred_element_type=jnp.float32) + b1_ref[...], 0.0)
    out_ref[...] = jnp.dot(h, w2_ref[...], preferred_element_type=jnp.float32) + b2_ref[...]


def _final_mlp(g, w1, b1, w2, b2):
    B = g.shape[0]
    OUT = w2.shape[1]
    return pl.pallas_call(
        _final_mlp_kernel,
        out_shape=jax.ShapeDtypeStruct((B, OUT), jnp.float32),
    )(g, w1, b1.reshape(1, -1), w2, b2.reshape(1, -1))


def kernel(x, edge_attr, params, edge_index, batch, fragments, fragments_edge_index, fragments_batch):
    p = params
    n = x.shape[0]
    f = fragments.shape[0]
    B = 256
    L = 4
    xa = x @ p['W_atom'] + p['b_atom']
    xf = p['frag_emb'][fragments]
    arow = fragments_edge_index[0]
    fcol = fragments_edge_index[1]
    ones_a = jnp.ones((arow.shape[0], 1), jnp.float32)
    rcnt_a = 1.0 / jnp.maximum(jax.ops.segment_sum(ones_a, arow, num_segments=n), 1.0)
    rcnt_f = 1.0 / jnp.maximum(jax.ops.segment_sum(ones_a, fcol, num_segments=f), 1.0)
    rcnt_b = 1.0 / jnp.maximum(jax.ops.segment_sum(jnp.ones((n, 1), jnp.float32), batch, num_segments=B), 1.0)
    rcnt_fb = 1.0 / jnp.maximum(jax.ops.segment_sum(jnp.ones((f, 1), jnp.float32), fragments_batch, num_segments=B), 1.0)
    xa = xa + _sc_gather_scatter(xf, fcol, arow, n) * rcnt_a
    src = edge_index[0]
    dst = edge_index[1]
    for i in range(L):
        e = edge_attr @ p['W_bond'][i] + p['b_bond'][i]
        parts = _sc_edge_agg(xa, e, src, dst)
        agg = parts[0] + parts[1]
        h = (1.0 + p['eps_a'][i]) * xa + agg
        h = h @ p['a_W1'][i] + p['a_b1'][i]
        h = jax.nn.relu(_bn(h, p['a_g1'][i], p['a_be1'][i]))
        h = h @ p['a_W2'][i] + p['a_b2'][i]
        xa = jax.nn.relu(_bn(h, p['a_gbn'][i], p['a_bbn'][i]))
        msg = jax.nn.relu(xa @ p['a2f_W'][i] + p['a2f_b'][i])
        xf = xf + _sc_gather_scatter(msg, arow, fcol, f) * rcnt_f
        hf = (1.0 + p['eps_f'][i]) * xf
        hf = hf @ p['f_W1'][i] + p['f_b1'][i]
        hf = jax.nn.relu(_bn(hf, p['f_g1'][i], p['f_be1'][i]))
        hf = hf @ p['f_W2'][i] + p['f_b2'][i]
        xf = jax.nn.relu(_bn(hf, p['f_gbn'][i], p['f_bbn'][i]))
        msg2 = jax.nn.relu(xf @ p['f2a_W'][i] + p['f2a_b'][i])
        xa = xa + _sc_gather_scatter(msg2, fcol, arow, n) * rcnt_a
    xf_o = xf
    for j in range(2):
        xf_o = jax.nn.relu(xf_o @ p['fo_W'][j] + p['fo_b'][j])
    xa_o = xa
    for j in range(2):
        xa_o = jax.nn.relu(xa_o @ p['ao_W'][j] + p['ao_b'][j])
    gf = _sc_gather_scatter(xf_o, jnp.arange(f, dtype=jnp.int32), fragments_batch, B) * rcnt_fb
    ga = _sc_gather_scatter(xa_o, jnp.arange(n, dtype=jnp.int32), batch, B) * rcnt_b
    g = ga + gf
    return _final_mlp(g, p['o_W1'], p['o_b1'], p['o_W2'], p['o_b2'])


# speedup vs baseline: 4.2585x; 1.4328x over previous
import functools

import jax
import jax.numpy as jnp
from jax import lax
from jax.experimental import pallas as pl
from jax.experimental.pallas import tpu as pltpu
from jax.experimental.pallas import tpu_sc as plsc

NC = 2
NS = 16
CHUNK = 128
ECHUNK = 64


def _sc_edge_kernel(xa_hbm, e_hbm, src_hbm, dst_hbm, out_hbm,
                    srcb, dstb, xbuf, ebuf, zbuf, acc,
                    sem_src, sem_dst, sem_g, sem_e):
    n_pad = acc.shape[0]
    n_edges = e_hbm.shape[0]
    total_chunks = n_edges // ECHUNK
    nworkers = NC * NS
    c = lax.axis_index("c")
    s = lax.axis_index("s")
    wid = s * NC + c

    def _zrow(i, _):
        for j in range(8):
            zbuf[i, pl.ds(j * 16, 16)] = jnp.zeros((16,), jnp.float32)
        return 0
    lax.fori_loop(0, ECHUNK, _zrow, 0)
    rows_per_tile = n_pad // NS
    base = s * rows_per_tile
    nfull = rows_per_tile // ECHUNK
    for k in range(nfull):
        pltpu.sync_copy(zbuf, acc.at[pl.ds(base + k * ECHUNK, ECHUNK)])
    rem = rows_per_tile - nfull * ECHUNK
    if rem:
        pltpu.sync_copy(zbuf.at[pl.ds(0, rem)],
                        acc.at[pl.ds(base + nfull * ECHUNK, rem)])
    plsc.subcore_barrier()

    n_my = (total_chunks - wid + nworkers - 1) // nworkers

    def _off(i):
        return (wid + i * nworkers) * ECHUNK

    def _issue_idx(i):
        slot = lax.rem(i, 2)
        off = _off(i)
        pltpu.async_copy(src_hbm.at[pl.ds(off, ECHUNK)], srcb.at[slot],
                         sem_src.at[slot])
        pltpu.async_copy(dst_hbm.at[pl.ds(off, ECHUNK)], dstb.at[slot],
                         sem_dst.at[slot])

    def _wait_idx(i):
        slot = lax.rem(i, 2)
        off = _off(i)
        pltpu.make_async_copy(src_hbm.at[pl.ds(off, ECHUNK)], srcb.at[slot],
                              sem_src.at[slot]).wait()
        pltpu.make_async_copy(dst_hbm.at[pl.ds(off, ECHUNK)], dstb.at[slot],
                              sem_dst.at[slot]).wait()

    def _issue_fetch(i):
        slot = lax.rem(i, 2)
        off = _off(i)
        pltpu.async_copy(xa_hbm.at[srcb.at[slot]], xbuf.at[slot],
                         sem_g.at[slot])
        pltpu.async_copy(e_hbm.at[pl.ds(off, ECHUNK)], ebuf.at[slot],
                         sem_e.at[slot])

    def _wait_fetch(i):
        slot = lax.rem(i, 2)
        off = _off(i)
        pltpu.make_async_copy(xa_hbm.at[srcb.at[slot]], xbuf.at[slot],
                              sem_g.at[slot]).wait()
        pltpu.make_async_copy(e_hbm.at[pl.ds(off, ECHUNK)], ebuf.at[slot],
                              sem_e.at[slot]).wait()

    @pl.when(n_my > 0)
    def _prologue():
        _issue_idx(0)

        @pl.when(n_my > 1)
        def _():
            _issue_idx(1)
        _wait_idx(0)
        _issue_fetch(0)

    def _chunk(i, _):
        slot = lax.rem(i, 2)

        @pl.when(i + 1 < n_my)
        def _():
            _wait_idx(i + 1)
            _issue_fetch(i + 1)
        _wait_fetch(i)

        @plsc.parallel_loop(0, ECHUNK, 1, unroll=4)
        def _row(r):
            for j in range(8):
                sl = pl.ds(j * 16, 16)
                xbuf[slot, r, sl] = jnp.maximum(
                    xbuf[slot, r, sl] + ebuf[slot, r, sl], 0.0)

        pltpu.sync_copy(xbuf.at[slot], acc.at[dstb.at[slot]], add=True)

        @pl.when(i + 2 < n_my)
        def _():
            _issue_idx(i + 2)
        return 0

    lax.fori_loop(0, n_my, _chunk, 0)
    plsc.subcore_barrier()
    pltpu.sync_copy(acc.at[pl.ds(base, rows_per_tile)],
                    out_hbm.at[c, pl.ds(base, rows_per_tile)])


def _round_up(v, m):
    return (v + m - 1) // m * m


def _sc_gs_kernel(vals_hbm, gidx_hbm, sidx_hbm, out_hbm,
                  gbuf, sbuf, xbuf, zbuf, acc):
    n_pad = acc.shape[0]
    k_pad = gidx_hbm.shape[0]
    total_chunks = k_pad // CHUNK
    nworkers = NC * NS
    c = lax.axis_index("c")
    s = lax.axis_index("s")
    wid = s * NC + c

    def _zrow(i, _):
        for j in range(8):
            zbuf[i, pl.ds(j * 16, 16)] = jnp.zeros((16,), jnp.float32)
        return 0
    lax.fori_loop(0, CHUNK, _zrow, 0)
    rows_per_tile = n_pad // NS
    base = s * rows_per_tile
    nfull = rows_per_tile // CHUNK
    for k in range(nfull):
        pltpu.sync_copy(zbuf, acc.at[pl.ds(base + k * CHUNK, CHUNK)])
    rem = rows_per_tile - nfull * CHUNK
    if rem:
        pltpu.sync_copy(zbuf.at[pl.ds(0, rem)],
                        acc.at[pl.ds(base + nfull * CHUNK, rem)])
    plsc.subcore_barrier()

    n_my = (total_chunks - wid + nworkers - 1) // nworkers

    def _chunk(i, _):
        off = (wid + i * nworkers) * CHUNK
        pltpu.sync_copy(gidx_hbm.at[pl.ds(off, CHUNK)], gbuf)
        pltpu.sync_copy(sidx_hbm.at[pl.ds(off, CHUNK)], sbuf)
        pltpu.sync_copy(vals_hbm.at[gbuf], xbuf)
        pltpu.sync_copy(xbuf, acc.at[sbuf], add=True)
        return 0

    lax.fori_loop(0, n_my, _chunk, 0)
    plsc.subcore_barrier()
    pltpu.sync_copy(acc.at[pl.ds(base, rows_per_tile)],
                    out_hbm.at[c, pl.ds(base, rows_per_tile)])


def _sc_gather_scatter(vals, gidx, sidx, r):
    h = vals.shape[1]
    k = gidx.shape[0]
    k_pad = _round_up(k, CHUNK)
    n_pad = _round_up(r + 1, NS * 8)
    if k_pad != k:
        gidx = jnp.concatenate([gidx, jnp.zeros((k_pad - k,), jnp.int32)])
        sidx = jnp.concatenate([sidx, jnp.full((k_pad - k,), r, jnp.int32)])
    mesh = plsc.VectorSubcoreMesh(core_axis_name="c", subcore_axis_name="s",
                                  num_cores=NC, num_subcores=NS)
    fn = pl.kernel(
        _sc_gs_kernel,
        out_type=jax.ShapeDtypeStruct((NC, n_pad, h), jnp.float32),
        mesh=mesh,
        scratch_types=[
            pltpu.VMEM((CHUNK,), jnp.int32),
            pltpu.VMEM((CHUNK,), jnp.int32),
            pltpu.VMEM((CHUNK, h), jnp.float32),
            pltpu.VMEM((CHUNK, h), jnp.float32),
            pltpu.VMEM_SHARED((n_pad, h), jnp.float32),
        ],
    )
    parts = fn(vals, gidx, sidx)
    return (parts[0] + parts[1])[:r]


def _sc_edge_agg(xa, e, src, dst):
    n = xa.shape[0]
    h = xa.shape[1]
    n_pad = _round_up(n, NS * 8)
    mesh = plsc.VectorSubcoreMesh(core_axis_name="c", subcore_axis_name="s",
                                  num_cores=NC, num_subcores=NS)
    fn = pl.kernel(
        _sc_edge_kernel,
        out_type=jax.ShapeDtypeStruct((NC, n_pad, h), jnp.float32),
        mesh=mesh,
        scratch_types=[
            pltpu.VMEM((2, ECHUNK), jnp.int32),
            pltpu.VMEM((2, ECHUNK), jnp.int32),
            pltpu.VMEM((2, ECHUNK, h), jnp.float32),
            pltpu.VMEM((2, ECHUNK, h), jnp.float32),
            pltpu.VMEM((ECHUNK, h), jnp.float32),
            pltpu.VMEM_SHARED((n_pad, h), jnp.float32),
            pltpu.SemaphoreType.DMA((2,)),
            pltpu.SemaphoreType.DMA((2,)),
            pltpu.SemaphoreType.DMA((2,)),
            pltpu.SemaphoreType.DMA((2,)),
        ],
    )
    return fn(xa, e, src, dst)[:, :n]


def _bn(h, g, b):
    mu = jnp.mean(h, axis=0)
    var = jnp.var(h, axis=0)
    return g * (h - mu) / jnp.sqrt(var + 1e-5) + b


def _seg_mean(vals, idx, num):
    s = jax.ops.segment_sum(vals, idx, num_segments=num)
    c = jax.ops.segment_sum(jnp.ones((vals.shape[0], 1), vals.dtype), idx, num_segments=num)
    return s / jnp.maximum(c, 1.0)


def _final_mlp_kernel(g_ref, w1_ref, b1_ref, w2_ref, b2_ref, out_ref):
    g = g_ref[...]
    h = jnp.maximum(jnp.dot(g, w1_ref[...], preferred_element_type=jnp.float32) + b1_ref[...], 0.0)
    out_ref[...] = jnp.dot(h, w2_ref[...], preferred_element_type=jnp.float32) + b2_ref[...]


def _final_mlp(g, w1, b1, w2, b2):
    B = g.shape[0]
    OUT = w2.shape[1]
    return pl.pallas_call(
        _final_mlp_kernel,
        out_shape=jax.ShapeDtypeStruct((B, OUT), jnp.float32),
    )(g, w1, b1.reshape(1, -1), w2, b2.reshape(1, -1))


def kernel(x, edge_attr, params, edge_index, batch, fragments, fragments_edge_index, fragments_batch):
    p = params
    n = x.shape[0]
    f = fragments.shape[0]
    B = 256
    L = 4
    xa = x @ p['W_atom'] + p['b_atom']
    xf = p['frag_emb'][fragments]
    arow = fragments_edge_index[0]
    fcol = fragments_edge_index[1]
    ones_a = jnp.ones((arow.shape[0], 1), jnp.float32)
    cnt_a = jnp.maximum(jax.ops.segment_sum(ones_a, arow, num_segments=n), 1.0)
    cnt_f = jnp.maximum(jax.ops.segment_sum(ones_a, fcol, num_segments=f), 1.0)
    cnt_b = jnp.maximum(jax.ops.segment_sum(jnp.ones((n, 1), jnp.float32), batch, num_segments=B), 1.0)
    cnt_fb = jnp.maximum(jax.ops.segment_sum(jnp.ones((f, 1), jnp.float32), fragments_batch, num_segments=B), 1.0)
    xa = xa + _sc_gather_scatter(xf, fcol, arow, n) / cnt_a
    src = edge_index[0]
    dst = edge_index[1]
    for i in range(L):
        e = edge_attr @ p['W_bond'][i] + p['b_bond'][i]
        parts = _sc_edge_agg(xa, e, src, dst)
        agg = parts[0] + parts[1]
        h = (1.0 + p['eps_a'][i]) * xa + agg
        h = h @ p['a_W1'][i] + p['a_b1'][i]
        h = jax.nn.relu(_bn(h, p['a_g1'][i], p['a_be1'][i]))
        h = h @ p['a_W2'][i] + p['a_b2'][i]
        xa = jax.nn.relu(_bn(h, p['a_gbn'][i], p['a_bbn'][i]))
        msg = jax.nn.relu(xa @ p['a2f_W'][i] + p['a2f_b'][i])
        xf = xf + _sc_gather_scatter(msg, arow, fcol, f) / cnt_f
        hf = (1.0 + p['eps_f'][i]) * xf
        hf = hf @ p['f_W1'][i] + p['f_b1'][i]
        hf = jax.nn.relu(_bn(hf, p['f_g1'][i], p['f_be1'][i]))
        hf = hf @ p['f_W2'][i] + p['f_b2'][i]
        xf = jax.nn.relu(_bn(hf, p['f_gbn'][i], p['f_bbn'][i]))
        msg2 = jax.nn.relu(xf @ p['f2a_W'][i] + p['f2a_b'][i])
        xa = xa + _sc_gather_scatter(msg2, fcol, arow, n) / cnt_a
    xf_o = xf
    for j in range(2):
        xf_o = jax.nn.relu(xf_o @ p['fo_W'][j] + p['fo_b'][j])
    xa_o = xa
    for j in range(2):
        xa_o = jax.nn.relu(xa_o @ p['ao_W'][j] + p['ao_b'][j])
    gf = _sc_gather_scatter(xf_o, jnp.arange(f, dtype=jnp.int32), fragments_batch, B) / cnt_fb
    ga = _sc_gather_scatter(xa_o, jnp.arange(n, dtype=jnp.int32), batch, B) / cnt_b
    g = ga + gf
    return _final_mlp(g, p['o_W1'], p['o_b1'], p['o_W2'], p['o_b2'])
